# baseline (speedup 1.0000x reference)
import functools

import jax
import jax.numpy as jnp
from jax import lax
from jax.experimental import pallas as pl
from jax.experimental.pallas import tpu as pltpu
from jax.experimental.pallas import tpu_sc as plsc

N_NODES = 10000
N_EDGES = 160000
D = 256
B = 1024
NB = 2 * B
MAPN = 10368
NC, NS = 2, 16
NW = NC * NS
E_PER_W = N_EDGES // NW
GROUPS = (E_PER_W + 15) // 16
AGG_ROWS = 2208
FIRE = 96
HITCAP = E_PER_W + FIRE + 16
TRASH = 2048
NB_PER_W = NB // NW

_i32 = jnp.int32
_f32 = jnp.float32

_MESH = plsc.VectorSubcoreMesh(core_axis_name="c", subcore_axis_name="s")
_SC_PARAMS = pltpu.CompilerParams(
    needs_layout_passes=False, use_tc_tiling_on_sc=False)


def _wid():
    return lax.axis_index("c") * NS + lax.axis_index("s")


def _map_body(n1_hbm, n2_hbm, map_out, perm_out, map_v, nodes_v, perm_v):
    @pl.when(_wid() == 0)
    def _():
        neg1 = jnp.full((16,), -1, _i32)

        def _zero(i, c):
            map_v[pl.ds(i * 16, 16)] = neg1
            return c
        lax.fori_loop(0, MAPN // 16, _zero, 0)

        pltpu.sync_copy(n1_hbm, nodes_v.at[pl.ds(0, B)])
        pltpu.sync_copy(n2_hbm, nodes_v.at[pl.ds(B, B)])

        def _scatter(i, c):
            idx = nodes_v[pl.ds(i * 16, 16)]
            val = lax.iota(_i32, 16) + i * 16
            plsc.store_scatter(map_v, [idx], val)
            return c
        lax.fori_loop(0, NB // 16, _scatter, 0)

        def _gather(i, c):
            idx = nodes_v[pl.ds(i * 16, 16)]
            perm_v[pl.ds(i * 16, 16)] = plsc.load_gather(map_v, [idx])
            return c
        lax.fori_loop(0, NB // 16, _gather, 0)

        pltpu.sync_copy(map_v, map_out)
        pltpu.sync_copy(perm_v, perm_out)


@functools.partial(
    pl.kernel,
    out_type=(
        jax.ShapeDtypeStruct((MAPN,), _i32),
        jax.ShapeDtypeStruct((NB,), _i32),
    ),
    mesh=_MESH,
    compiler_params=_SC_PARAMS,
    scratch_types=[
        pltpu.VMEM((MAPN,), _i32),
        pltpu.VMEM((NB,), _i32),
        pltpu.VMEM((NB,), _i32),
    ],
)
def _map_kernel(n1_hbm, n2_hbm, map_out, perm_out, map_v, nodes_v, perm_v):
    _map_body(n1_hbm, n2_hbm, map_out, perm_out, map_v, nodes_v, perm_v)


def _edge_body(x_hbm, ei_hbm, map_hbm, n1_hbm, n2_hbm,
               agg_out, deg_out, xb_out,
               map_v, src_v, dst_v, hit_slot, hit_src,
               src_fire, slot_fire, src_fire2, slot_fire2,
               rows_v, rows_v2, ones_v, idx_v,
               gsem0, gsem1, ssem0, ssem1, dsem0, dsem1,
               agg_sh, deg_sh):
    cid = lax.axis_index("c")
    sid = lax.axis_index("s")
    wid = cid * NS + sid

    zeros16 = jnp.zeros((16,), _i32)
    dst_v[pl.ds(E_PER_W - 8, 16)] = zeros16
    map_cp = pltpu.make_async_copy(map_hbm, map_v, gsem0)
    src_cp = pltpu.make_async_copy(
        ei_hbm.at[0, pl.ds(wid * E_PER_W, E_PER_W)],
        src_v.at[pl.ds(0, E_PER_W)], gsem1)
    dst_cp = pltpu.make_async_copy(
        ei_hbm.at[1, pl.ds(wid * E_PER_W, E_PER_W)],
        dst_v.at[pl.ds(0, E_PER_W)], ssem0)
    map_cp.start()
    src_cp.start()
    dst_cp.start()

    zf = jnp.zeros((16,), _f32)

    def _zrows(i, c):
        r = i // 16
        col = (i % 16) * 16
        rows_v[r, pl.ds(col, 16)] = zf
        return c
    lax.fori_loop(0, FIRE * 16, _zrows, 0)

    def _zones(i, c):
        ones_v[i, pl.ds(0, 16)] = zf
        return c
    lax.fori_loop(0, FIRE, _zones, 0)

    map_cp.wait()
    src_cp.wait()
    dst_cp.wait()

    NZ = AGG_ROWS // FIRE
    za = pltpu.make_async_copy(
        rows_v, agg_sh.at[pl.ds(sid * FIRE, FIRE)], ssem1)
    zb = pltpu.make_async_copy(
        ones_v, deg_sh.at[pl.ds(sid * FIRE, FIRE)], dsem0)
    za.start()
    zb.start()

    @pl.when(sid < NZ - NS)
    def _():
        pltpu.make_async_copy(
            rows_v, agg_sh.at[pl.ds((NS + sid) * FIRE, FIRE)], dsem1).start()
        pltpu.make_async_copy(
            ones_v, deg_sh.at[pl.ds((NS + sid) * FIRE, FIRE)], gsem0).start()

    za.wait()
    zb.wait()

    @pl.when(sid < NZ - NS)
    def _():
        pltpu.make_async_copy(
            rows_v, agg_sh.at[pl.ds((NS + sid) * FIRE, FIRE)], dsem1).wait()
        pltpu.make_async_copy(
            ones_v, deg_sh.at[pl.ds((NS + sid) * FIRE, FIRE)], gsem0).wait()

    one0 = jnp.where(lax.iota(_i32, 16) == 0, 1.0, 0.0).astype(_f32)

    def _ones(i, c):
        ones_v[i, pl.ds(0, 16)] = one0
        return c
    lax.fori_loop(0, FIRE, _ones, 0)

    plsc.subcore_barrier()

    _scope_xb = jax.named_scope("xb_gather")
    _scope_xb.__enter__()
    @pl.when(wid < NS)
    def _():
        pltpu.sync_copy(n1_hbm.at[pl.ds(wid * NB_PER_W, NB_PER_W)], idx_v)

    @pl.when(wid >= NS)
    def _():
        pltpu.sync_copy(n2_hbm.at[pl.ds((wid - NS) * NB_PER_W, NB_PER_W)],
                        idx_v)
    pltpu.sync_copy(x_hbm.at[plsc.Indices(idx_v)],
                    rows_v.at[pl.ds(0, NB_PER_W)])
    pltpu.sync_copy(rows_v.at[pl.ds(0, NB_PER_W)],
                    xb_out.at[pl.ds(wid * NB_PER_W, NB_PER_W)])

    _scope_xb.__exit__(None, None, None)
    lanes = lax.iota(_i32, 16)

    _scope_compact = jax.named_scope("phaseA_compact")
    _scope_compact.__enter__()

    def _compact(g, cnt):
        base = g * 16
        dvec = dst_v[pl.ds(base, 16)]
        svec = src_v[pl.ds(base, 16)]
        slots = plsc.load_gather(map_v, [dvec])
        hit = slots >= 0
        plsc.store_compressed(hit_slot.at[pl.ds(cnt, 16)], slots, mask=hit)
        plsc.store_compressed(hit_src.at[pl.ds(cnt, 16)], svec, mask=hit)
        return cnt + plsc.all_reduce_population_count(hit)[0]

    cnt = lax.fori_loop(0, GROUPS - 1, _compact, jnp.int32(0))

    tbase = (GROUPS - 1) * 16
    tslots = plsc.load_gather(map_v, [dst_v[pl.ds(tbase, 16)]])
    thit = (tslots >= 0) & (lanes < E_PER_W - tbase)
    plsc.store_compressed(hit_slot.at[pl.ds(cnt, 16)], tslots, mask=thit)
    plsc.store_compressed(hit_src.at[pl.ds(cnt, 16)],
                          src_v[pl.ds(tbase, 16)], mask=thit)
    cnt = cnt + plsc.all_reduce_population_count(thit)[0]

    trash = jnp.full((16,), TRASH, _i32)
    for j in range(FIRE // 16):
        hit_slot[pl.ds(cnt + j * 16, 16)] = trash
        hit_src[pl.ds(cnt + j * 16, 16)] = zeros16

    _scope_compact.__exit__(None, None, None)
    _scope_fire = jax.named_scope("phaseB_fire")
    _scope_fire.__enter__()

    n_chunks = (cnt + FIRE - 1) // FIRE
    rows = (rows_v, rows_v2)
    src_f = (src_fire, src_fire2)
    slot_f = (slot_fire, slot_fire2)
    gsem = (gsem0, gsem1)
    ssem = (ssem0, ssem1)
    dsem = (dsem0, dsem1)

    def _ga(p):
        return pltpu.make_async_copy(
            x_hbm.at[plsc.Indices(src_f[p])], rows[p], gsem[p])

    def _sa(p):
        return pltpu.make_async_copy(
            rows[p], agg_sh.at[plsc.Indices(slot_f[p])], ssem[p])

    def _da(p):
        return pltpu.make_async_copy(
            ones_v, deg_sh.at[plsc.Indices(slot_f[p])], dsem[p])

    def _load_idx(p, k):
        base = k * FIRE
        for j in range(FIRE // 16):
            src_f[p][pl.ds(j * 16, 16)] = hit_src[pl.ds(base + j * 16, 16)]
            slot_f[p][pl.ds(j * 16, 16)] = hit_slot[pl.ds(base + j * 16, 16)]

    @pl.when(n_chunks > 0)
    def _():
        _load_idx(0, 0)
        _ga(0).start()
        _da(0).start(add=True)

    def _step(k, p):
        def _one(p):
            q = 1 - p
            _ga(p).wait()
            _sa(p).start(add=True)

            @pl.when(k + 1 < n_chunks)
            def _():
                @pl.when(k >= 1)
                def _():
                    _sa(q).wait()
                    _da(q).wait()
                _load_idx(q, k + 1)
                _ga(q).start()
                _da(q).start(add=True)

        @pl.when(p == 0)
        def _():
            _one(0)

        @pl.when(p == 1)
        def _():
            _one(1)

        return 1 - p

    lax.fori_loop(0, n_chunks, _step, jnp.int32(0))

    last = n_chunks - 1

    @pl.when(n_chunks >= 2)
    def _():
        p = (last - 1) % 2

        @pl.when(p == 0)
        def _():
            _sa(0).wait()
            _da(0).wait()

        @pl.when(p == 1)
        def _():
            _sa(1).wait()
            _da(1).wait()

    @pl.when(n_chunks >= 1)
    def _():
        p = last % 2

        @pl.when(p == 0)
        def _():
            _sa(0).wait()
            _da(0).wait()

        @pl.when(p == 1)
        def _():
            _sa(1).wait()
            _da(1).wait()

    _scope_fire.__exit__(None, None, None)
    plsc.subcore_barrier()

    RPW = AGG_ROWS // NS
    co_a = pltpu.make_async_copy(agg_sh.at[pl.ds(sid * RPW, RPW)],
                                 agg_out.at[cid, pl.ds(sid * RPW, RPW)],
                                 gsem0)
    co_d = pltpu.make_async_copy(deg_sh.at[pl.ds(sid * RPW, RPW)],
                                 deg_out.at[cid, pl.ds(sid * RPW, RPW)],
                                 gsem1)
    co_a.start()
    co_d.start()
    co_a.wait()
    co_d.wait()


@functools.partial(
    pl.kernel,
    out_type=(
        jax.ShapeDtypeStruct((NC, AGG_ROWS, D), _f32),
        jax.ShapeDtypeStruct((NC, AGG_ROWS, 16), _f32),
        jax.ShapeDtypeStruct((NB, D), _f32),
    ),
    mesh=_MESH,
    compiler_params=_SC_PARAMS,
    scratch_types=[
        pltpu.VMEM((MAPN,), _i32),
        pltpu.VMEM((E_PER_W + 8,), _i32),
        pltpu.VMEM((E_PER_W + 8,), _i32),
        pltpu.VMEM((HITCAP,), _i32),
        pltpu.VMEM((HITCAP,), _i32),
        pltpu.VMEM((FIRE,), _i32),
        pltpu.VMEM((FIRE,), _i32),
        pltpu.VMEM((FIRE,), _i32),
        pltpu.VMEM((FIRE,), _i32),
        pltpu.VMEM((FIRE, D), _f32),
        pltpu.VMEM((FIRE, D), _f32),
        pltpu.VMEM((FIRE, 16), _f32),
        pltpu.VMEM((NB_PER_W,), _i32),
        pltpu.SemaphoreType.DMA,
        pltpu.SemaphoreType.DMA,
        pltpu.SemaphoreType.DMA,
        pltpu.SemaphoreType.DMA,
        pltpu.SemaphoreType.DMA,
        pltpu.SemaphoreType.DMA,
        pltpu.VMEM_SHARED((AGG_ROWS, D), _f32),
        pltpu.VMEM_SHARED((AGG_ROWS, 16), _f32),
    ],
)
def _edge_kernel(*refs):
    _edge_body(*refs)


def _dense_body(xb_ref, agg_ref, deg_ref, ws_ref, wn_ref, h_ref):
    agg = agg_ref[0, :NB, :] + agg_ref[1, :NB, :]
    deg = jnp.sum(deg_ref[0, :NB, :] + deg_ref[1, :NB, :], axis=1,
                  keepdims=True)
    mean = agg / jnp.maximum(deg, 1.0)
    h = jnp.dot(xb_ref[...], ws_ref[...],
                preferred_element_type=_f32,
                precision=lax.Precision.HIGHEST)
    h = h + jnp.dot(mean, wn_ref[...],
                    preferred_element_type=_f32,
                    precision=lax.Precision.HIGHEST)
    h = jnp.maximum(h, 0.0)
    norm = jnp.sqrt(jnp.sum(h * h, axis=1, keepdims=True))
    h_ref[...] = h / jnp.maximum(norm, 1e-12)


def _dense_call(xb, agg, deg, w_self, w_neigh):
    return pl.pallas_call(
        _dense_body,
        out_shape=jax.ShapeDtypeStruct((NB, D), _f32),
    )(xb, agg, deg, w_self, w_neigh)


def _out_body(h_hbm, perm_hbm, e1_out, e2_out, idx_v, rows_v):
    wid = _wid()
    pltpu.sync_copy(perm_hbm.at[pl.ds(wid * NB_PER_W, NB_PER_W)], idx_v)
    pltpu.sync_copy(h_hbm.at[plsc.Indices(idx_v)], rows_v)

    @pl.when(wid < NS)
    def _():
        pltpu.sync_copy(rows_v, e1_out.at[pl.ds(wid * NB_PER_W, NB_PER_W)])

    @pl.when(wid >= NS)
    def _():
        pltpu.sync_copy(rows_v,
                        e2_out.at[pl.ds((wid - NS) * NB_PER_W, NB_PER_W)])


@functools.partial(
    pl.kernel,
    out_type=(
        jax.ShapeDtypeStruct((B, D), _f32),
        jax.ShapeDtypeStruct((B, D), _f32),
    ),
    mesh=_MESH,
    compiler_params=_SC_PARAMS,
    scratch_types=[
        pltpu.VMEM((NB_PER_W,), _i32),
        pltpu.VMEM((NB_PER_W, D), _f32),
    ],
)
def _out_kernel(h_hbm, perm_hbm, e1_out, e2_out, idx_v, rows_v):
    _out_body(h_hbm, perm_hbm, e1_out, e2_out, idx_v, rows_v)


def kernel(x, edge_index, nodes1, nodes2, W_self, W_neigh):
    map_arr, perm = _map_kernel(nodes1, nodes2)
    agg, deg, xb = _edge_kernel(x, edge_index, map_arr, nodes1, nodes2)
    h = _dense_call(xb, agg, deg, W_self, W_neigh)
    e1, e2 = _out_kernel(h, perm)
    return (e1, e2)


# speedup vs baseline: 1.0384x; 1.0384x over previous
import functools

import jax
import jax.numpy as jnp
from jax import lax
from jax.experimental import pallas as pl
from jax.experimental.pallas import tpu as pltpu
from jax.experimental.pallas import tpu_sc as plsc

N_NODES = 10000
N_EDGES = 160000
D = 256
B = 1024
NB = 2 * B
MAPN = 10368
NC, NS = 2, 16
NW = NC * NS
E_PER_W = N_EDGES // NW
GROUPS = (E_PER_W + 15) // 16
AGG_ROWS = 2112
FIRE = 96
HITCAP = E_PER_W + FIRE + 16
TRASH = 2048
NB_PER_W = NB // NW

_i32 = jnp.int32
_f32 = jnp.float32

_MESH = plsc.VectorSubcoreMesh(core_axis_name="c", subcore_axis_name="s")
_SC_PARAMS = pltpu.CompilerParams(
    needs_layout_passes=False, use_tc_tiling_on_sc=False)


def _edge_body(x_hbm, ei_hbm, n1_hbm, n2_hbm,
               agg_out, deg_out, xb_out, perm_out,
               map_v, nodes_v, src_v, dst_v, hit_slot, hit_src,
               src_fire, slot_fire, src_fire2, slot_fire2,
               rows_v, rows_v2, ones_v, idx_v,
               gsem0, gsem1, ssem0, ssem1, dsem0, dsem1,
               agg_sh, deg_sh):
    cid = lax.axis_index("c")
    sid = lax.axis_index("s")
    wid = cid * NS + sid

    zeros16 = jnp.zeros((16,), _i32)
    dst_v[pl.ds(E_PER_W - 8, 16)] = zeros16
    n1_cp = pltpu.make_async_copy(n1_hbm, nodes_v.at[pl.ds(0, B)], gsem0)
    n2_cp = pltpu.make_async_copy(n2_hbm, nodes_v.at[pl.ds(B, B)], dsem1)
    src_cp = pltpu.make_async_copy(
        ei_hbm.at[0, pl.ds(wid * E_PER_W, E_PER_W)],
        src_v.at[pl.ds(0, E_PER_W)], gsem1)
    dst_cp = pltpu.make_async_copy(
        ei_hbm.at[1, pl.ds(wid * E_PER_W, E_PER_W)],
        dst_v.at[pl.ds(0, E_PER_W)], ssem0)
    n1_cp.start()
    n2_cp.start()
    src_cp.start()
    dst_cp.start()

    neg1 = jnp.full((16,), -1, _i32)

    def _zmap(i, c):
        map_v[pl.ds(i * 16, 16)] = neg1
        return c
    lax.fori_loop(0, MAPN // 16, _zmap, 0)

    zf = jnp.zeros((16,), _f32)

    def _zrows(i, c):
        r = i // 16
        col = (i % 16) * 16
        rows_v[r, pl.ds(col, 16)] = zf
        return c
    lax.fori_loop(0, FIRE * 16, _zrows, 0)

    def _zones(i, c):
        ones_v[i, pl.ds(0, 16)] = zf
        return c
    lax.fori_loop(0, FIRE, _zones, 0)

    n1_cp.wait()
    n2_cp.wait()

    def _scatter(i, c):
        idx = nodes_v[pl.ds(i * 16, 16)]
        val = lax.iota(_i32, 16) + i * 16
        plsc.store_scatter(map_v, [idx], val)
        return c
    lax.fori_loop(0, NB // 16, _scatter, 0)

    @pl.when(wid == 0)
    def _():
        def _gather(i, c):
            idx = nodes_v[pl.ds(i * 16, 16)]
            nodes_v[pl.ds(i * 16, 16)] = plsc.load_gather(map_v, [idx])
            return c
        lax.fori_loop(0, NB // 16, _gather, 0)
        pltpu.sync_copy(nodes_v, perm_out)

    NZ = AGG_ROWS // FIRE
    za = pltpu.make_async_copy(
        rows_v, agg_sh.at[pl.ds(sid * FIRE, FIRE)], ssem1)
    zb = pltpu.make_async_copy(
        ones_v, deg_sh.at[pl.ds(sid * FIRE, FIRE)], dsem0)
    za.start()
    zb.start()

    @pl.when(sid < NZ - NS)
    def _():
        pltpu.make_async_copy(
            rows_v, agg_sh.at[pl.ds((NS + sid) * FIRE, FIRE)], dsem1).start()
        pltpu.make_async_copy(
            ones_v, deg_sh.at[pl.ds((NS + sid) * FIRE, FIRE)], gsem0).start()

    za.wait()
    zb.wait()

    @pl.when(sid < NZ - NS)
    def _():
        pltpu.make_async_copy(
            rows_v, agg_sh.at[pl.ds((NS + sid) * FIRE, FIRE)], dsem1).wait()
        pltpu.make_async_copy(
            ones_v, deg_sh.at[pl.ds((NS + sid) * FIRE, FIRE)], gsem0).wait()

    one0 = jnp.where(lax.iota(_i32, 16) == 0, 1.0, 0.0).astype(_f32)

    def _ones(i, c):
        ones_v[i, pl.ds(0, 16)] = one0
        return c
    lax.fori_loop(0, FIRE, _ones, 0)

    src_cp.wait()
    dst_cp.wait()

    plsc.subcore_barrier()

    @pl.when(wid < NS)
    def _():
        pltpu.sync_copy(n1_hbm.at[pl.ds(wid * NB_PER_W, NB_PER_W)], idx_v)

    @pl.when(wid >= NS)
    def _():
        pltpu.sync_copy(n2_hbm.at[pl.ds((wid - NS) * NB_PER_W, NB_PER_W)],
                        idx_v)
    pltpu.sync_copy(x_hbm.at[plsc.Indices(idx_v)],
                    rows_v.at[pl.ds(0, NB_PER_W)])
    pltpu.sync_copy(rows_v.at[pl.ds(0, NB_PER_W)],
                    xb_out.at[pl.ds(wid * NB_PER_W, NB_PER_W)])

    lanes = lax.iota(_i32, 16)

    def _compact(g, cnt):
        base = g * 16
        dvec = dst_v[pl.ds(base, 16)]
        svec = src_v[pl.ds(base, 16)]
        slots = plsc.load_gather(map_v, [dvec])
        hit = slots >= 0
        plsc.store_compressed(hit_slot.at[pl.ds(cnt, 16)], slots, mask=hit)
        plsc.store_compressed(hit_src.at[pl.ds(cnt, 16)], svec, mask=hit)
        return cnt + plsc.all_reduce_population_count(hit)[0]

    cnt = lax.fori_loop(0, GROUPS - 1, _compact, jnp.int32(0))

    tbase = (GROUPS - 1) * 16
    tslots = plsc.load_gather(map_v, [dst_v[pl.ds(tbase, 16)]])
    thit = (tslots >= 0) & (lanes < E_PER_W - tbase)
    plsc.store_compressed(hit_slot.at[pl.ds(cnt, 16)], tslots, mask=thit)
    plsc.store_compressed(hit_src.at[pl.ds(cnt, 16)],
                          src_v[pl.ds(tbase, 16)], mask=thit)
    cnt = cnt + plsc.all_reduce_population_count(thit)[0]

    trash = jnp.full((16,), TRASH, _i32)
    for j in range(FIRE // 16):
        hit_slot[pl.ds(cnt + j * 16, 16)] = trash
        hit_src[pl.ds(cnt + j * 16, 16)] = zeros16

    n_chunks = (cnt + FIRE - 1) // FIRE
    rows = (rows_v, rows_v2)
    src_f = (src_fire, src_fire2)
    slot_f = (slot_fire, slot_fire2)
    gsem = (gsem0, gsem1)
    ssem = (ssem0, ssem1)
    dsem = (dsem0, dsem1)

    def _ga(p):
        return pltpu.make_async_copy(
            x_hbm.at[plsc.Indices(src_f[p])], rows[p], gsem[p])

    def _sa(p):
        return pltpu.make_async_copy(
            rows[p], agg_sh.at[plsc.Indices(slot_f[p])], ssem[p])

    def _da(p):
        return pltpu.make_async_copy(
            ones_v, deg_sh.at[plsc.Indices(slot_f[p])], dsem[p])

    def _load_idx(p, k):
        base = k * FIRE
        for j in range(FIRE // 16):
            src_f[p][pl.ds(j * 16, 16)] = hit_src[pl.ds(base + j * 16, 16)]
            slot_f[p][pl.ds(j * 16, 16)] = hit_slot[pl.ds(base + j * 16, 16)]

    @pl.when(n_chunks > 0)
    def _():
        _load_idx(0, 0)
        _ga(0).start()
        _da(0).start(add=True)

    def _step(k, p):
        def _one(p):
            q = 1 - p
            _ga(p).wait()
            _sa(p).start(add=True)

            @pl.when(k + 1 < n_chunks)
            def _():
                @pl.when(k >= 1)
                def _():
                    _sa(q).wait()
                    _da(q).wait()
                _load_idx(q, k + 1)
                _ga(q).start()
                _da(q).start(add=True)

        @pl.when(p == 0)
        def _():
            _one(0)

        @pl.when(p == 1)
        def _():
            _one(1)

        return 1 - p

    lax.fori_loop(0, n_chunks, _step, jnp.int32(0))

    last = n_chunks - 1

    @pl.when(n_chunks >= 2)
    def _():
        p = (last - 1) % 2

        @pl.when(p == 0)
        def _():
            _sa(0).wait()
            _da(0).wait()

        @pl.when(p == 1)
        def _():
            _sa(1).wait()
            _da(1).wait()

    @pl.when(n_chunks >= 1)
    def _():
        p = last % 2

        @pl.when(p == 0)
        def _():
            _sa(0).wait()
            _da(0).wait()

        @pl.when(p == 1)
        def _():
            _sa(1).wait()
            _da(1).wait()

    plsc.subcore_barrier()

    RPW = NB // NS
    co_a = pltpu.make_async_copy(agg_sh.at[pl.ds(sid * RPW, RPW)],
                                 agg_out.at[cid, pl.ds(sid * RPW, RPW)],
                                 gsem0)
    co_d = pltpu.make_async_copy(deg_sh.at[pl.ds(sid * RPW, RPW)],
                                 deg_out.at[cid, pl.ds(sid * RPW, RPW)],
                                 gsem1)
    co_a.start()
    co_d.start()
    co_a.wait()
    co_d.wait()


@functools.partial(
    pl.kernel,
    out_type=(
        jax.ShapeDtypeStruct((NC, NB, D), _f32),
        jax.ShapeDtypeStruct((NC, NB, 16), _f32),
        jax.ShapeDtypeStruct((NB, D), _f32),
        jax.ShapeDtypeStruct((NB,), _i32),
    ),
    mesh=_MESH,
    compiler_params=_SC_PARAMS,
    scratch_types=[
        pltpu.VMEM((MAPN,), _i32),
        pltpu.VMEM((NB,), _i32),
        pltpu.VMEM((E_PER_W + 8,), _i32),
        pltpu.VMEM((E_PER_W + 8,), _i32),
        pltpu.VMEM((HITCAP,), _i32),
        pltpu.VMEM((HITCAP,), _i32),
        pltpu.VMEM((FIRE,), _i32),
        pltpu.VMEM((FIRE,), _i32),
        pltpu.VMEM((FIRE,), _i32),
        pltpu.VMEM((FIRE,), _i32),
        pltpu.VMEM((FIRE, D), _f32),
        pltpu.VMEM((FIRE, D), _f32),
        pltpu.VMEM((FIRE, 16), _f32),
        pltpu.VMEM((NB_PER_W,), _i32),
        pltpu.SemaphoreType.DMA,
        pltpu.SemaphoreType.DMA,
        pltpu.SemaphoreType.DMA,
        pltpu.SemaphoreType.DMA,
        pltpu.SemaphoreType.DMA,
        pltpu.SemaphoreType.DMA,
        pltpu.VMEM_SHARED((AGG_ROWS, D), _f32),
        pltpu.VMEM_SHARED((AGG_ROWS, 16), _f32),
    ],
)
def _edge_kernel(*refs):
    _edge_body(*refs)


def _dense_body(xb_ref, agg_ref, deg_ref, ws_ref, wn_ref, h_ref):
    agg = agg_ref[0] + agg_ref[1]
    deg = jnp.sum(deg_ref[0] + deg_ref[1], axis=1, keepdims=True)
    mean = agg / jnp.maximum(deg, 1.0)
    h = jnp.dot(xb_ref[...], ws_ref[...],
                preferred_element_type=_f32,
                precision=lax.Precision.HIGHEST)
    h = h + jnp.dot(mean, wn_ref[...],
                    preferred_element_type=_f32,
                    precision=lax.Precision.HIGHEST)
    h = jnp.maximum(h, 0.0)
    norm = jnp.sqrt(jnp.sum(h * h, axis=1, keepdims=True))
    h_ref[...] = h / jnp.maximum(norm, 1e-12)


def _dense_call(xb, agg, deg, w_self, w_neigh):
    return pl.pallas_call(
        _dense_body,
        out_shape=jax.ShapeDtypeStruct((NB, D), _f32),
    )(xb, agg, deg, w_self, w_neigh)


def _out_body(h_hbm, perm_hbm, e1_out, e2_out, idx_v, rows_v):
    wid = lax.axis_index("c") * NS + lax.axis_index("s")
    pltpu.sync_copy(perm_hbm.at[pl.ds(wid * NB_PER_W, NB_PER_W)], idx_v)
    pltpu.sync_copy(h_hbm.at[plsc.Indices(idx_v)], rows_v)

    @pl.when(wid < NS)
    def _():
        pltpu.sync_copy(rows_v, e1_out.at[pl.ds(wid * NB_PER_W, NB_PER_W)])

    @pl.when(wid >= NS)
    def _():
        pltpu.sync_copy(rows_v,
                        e2_out.at[pl.ds((wid - NS) * NB_PER_W, NB_PER_W)])


@functools.partial(
    pl.kernel,
    out_type=(
        jax.ShapeDtypeStruct((B, D), _f32),
        jax.ShapeDtypeStruct((B, D), _f32),
    ),
    mesh=_MESH,
    compiler_params=_SC_PARAMS,
    scratch_types=[
        pltpu.VMEM((NB_PER_W,), _i32),
        pltpu.VMEM((NB_PER_W, D), _f32),
    ],
)
def _out_kernel(h_hbm, perm_hbm, e1_out, e2_out, idx_v, rows_v):
    _out_body(h_hbm, perm_hbm, e1_out, e2_out, idx_v, rows_v)


def kernel(x, edge_index, nodes1, nodes2, W_self, W_neigh):
    agg, deg, xb, perm = _edge_kernel(x, edge_index, nodes1, nodes2)
    h = _dense_call(xb, agg, deg, W_self, W_neigh)
    e1, e2 = _out_kernel(h, perm)
    return (e1, e2)


# speedup vs baseline: 1.0598x; 1.0206x over previous
import functools

import jax
import jax.numpy as jnp
from jax import lax
from jax.experimental import pallas as pl
from jax.experimental.pallas import tpu as pltpu
from jax.experimental.pallas import tpu_sc as plsc

N_NODES = 10000
N_EDGES = 160000
D = 256
B = 1024
NB = 2 * B
MAPN = 10368
NC, NS = 2, 16
NW = NC * NS
E_PER_W = N_EDGES // NW
GROUPS = (E_PER_W + 15) // 16
AGG_ROWS = 2112
FIRE = 96
HITCAP = E_PER_W + FIRE + 16
TRASH = 2048
NB_PER_W = NB // NW

_i32 = jnp.int32
_f32 = jnp.float32

_MESH = plsc.VectorSubcoreMesh(core_axis_name="c", subcore_axis_name="s")
_SC_PARAMS = pltpu.CompilerParams(
    needs_layout_passes=False, use_tc_tiling_on_sc=False)


def _edge_body(x_hbm, ei_hbm, n1_hbm, n2_hbm,
               agg_out, deg_out, xb_out, perm_out,
               map_v, nodes_v, src_v, dst_v, hit_slot, hit_src,
               src_fire, slot_fire, src_fire2, slot_fire2,
               rows_v, rows_v2, ones_v, idx_v,
               gsem0, gsem1, ssem0, ssem1, dsem0, dsem1,
               agg_sh, deg_sh):
    cid = lax.axis_index("c")
    sid = lax.axis_index("s")
    wid = cid * NS + sid

    zeros16 = jnp.zeros((16,), _i32)
    dst_v[pl.ds(E_PER_W - 8, 16)] = zeros16
    n1_cp = pltpu.make_async_copy(n1_hbm, nodes_v.at[pl.ds(0, B)], gsem0)
    n2_cp = pltpu.make_async_copy(n2_hbm, nodes_v.at[pl.ds(B, B)], dsem1)
    src_cp = pltpu.make_async_copy(
        ei_hbm.at[0, pl.ds(wid * E_PER_W, E_PER_W)],
        src_v.at[pl.ds(0, E_PER_W)], gsem1)
    dst_cp = pltpu.make_async_copy(
        ei_hbm.at[1, pl.ds(wid * E_PER_W, E_PER_W)],
        dst_v.at[pl.ds(0, E_PER_W)], ssem0)
    n1_cp.start()
    n2_cp.start()
    src_cp.start()
    dst_cp.start()

    neg1 = jnp.full((16,), -1, _i32)

    def _zmap(i, c):
        map_v[pl.ds(i * 16, 16)] = neg1
        return c
    lax.fori_loop(0, MAPN // 16, _zmap, 0)

    zf = jnp.zeros((16,), _f32)

    def _zrows(i, c):
        r = i // 16
        col = (i % 16) * 16
        rows_v[r, pl.ds(col, 16)] = zf
        return c
    lax.fori_loop(0, FIRE * 16, _zrows, 0)

    def _zones(i, c):
        ones_v[i, pl.ds(0, 16)] = zf
        return c
    lax.fori_loop(0, FIRE, _zones, 0)

    n1_cp.wait()
    n2_cp.wait()

    def _scatter(i, c):
        idx = nodes_v[pl.ds(i * 16, 16)]
        val = lax.iota(_i32, 16) + i * 16
        plsc.store_scatter(map_v, [idx], val)
        return c
    lax.fori_loop(0, NB // 16, _scatter, 0)

    @pl.when(wid == 0)
    def _():
        def _gather(i, c):
            idx = nodes_v[pl.ds(i * 16, 16)]
            nodes_v[pl.ds(i * 16, 16)] = plsc.load_gather(map_v, [idx])
            return c
        lax.fori_loop(0, NB // 16, _gather, 0)
        pltpu.sync_copy(nodes_v, perm_out)

    NZ = AGG_ROWS // FIRE
    za = pltpu.make_async_copy(
        rows_v, agg_sh.at[pl.ds(sid * FIRE, FIRE)], ssem1)
    zb = pltpu.make_async_copy(
        ones_v, deg_sh.at[pl.ds(sid * FIRE, FIRE)], dsem0)
    za.start()
    zb.start()

    @pl.when(sid < NZ - NS)
    def _():
        pltpu.make_async_copy(
            rows_v, agg_sh.at[pl.ds((NS + sid) * FIRE, FIRE)], dsem1).start()
        pltpu.make_async_copy(
            ones_v, deg_sh.at[pl.ds((NS + sid) * FIRE, FIRE)], gsem0).start()

    za.wait()
    zb.wait()

    @pl.when(sid < NZ - NS)
    def _():
        pltpu.make_async_copy(
            rows_v, agg_sh.at[pl.ds((NS + sid) * FIRE, FIRE)], dsem1).wait()
        pltpu.make_async_copy(
            ones_v, deg_sh.at[pl.ds((NS + sid) * FIRE, FIRE)], gsem0).wait()

    one0 = jnp.where(lax.iota(_i32, 16) == 0, 1.0, 0.0).astype(_f32)

    def _ones(i, c):
        ones_v[i, pl.ds(0, 16)] = one0
        return c
    lax.fori_loop(0, FIRE, _ones, 0)

    src_cp.wait()
    dst_cp.wait()

    plsc.subcore_barrier()

    @pl.when(wid < NS)
    def _():
        pltpu.sync_copy(n1_hbm.at[pl.ds(wid * NB_PER_W, NB_PER_W)], idx_v)

    @pl.when(wid >= NS)
    def _():
        pltpu.sync_copy(n2_hbm.at[pl.ds((wid - NS) * NB_PER_W, NB_PER_W)],
                        idx_v)
    pltpu.sync_copy(x_hbm.at[plsc.Indices(idx_v)],
                    rows_v.at[pl.ds(0, NB_PER_W)])
    pltpu.sync_copy(rows_v.at[pl.ds(0, NB_PER_W)],
                    xb_out.at[pl.ds(wid * NB_PER_W, NB_PER_W)])

    lanes = lax.iota(_i32, 16)

    def _compact(g, cnt):
        base = g * 16
        dvec = dst_v[pl.ds(base, 16)]
        svec = src_v[pl.ds(base, 16)]
        slots = plsc.load_gather(map_v, [dvec])
        hit = slots >= 0
        plsc.store_compressed(hit_slot.at[pl.ds(cnt, 16)], slots, mask=hit)
        plsc.store_compressed(hit_src.at[pl.ds(cnt, 16)], svec, mask=hit)
        return cnt + plsc.all_reduce_population_count(hit)[0]

    cnt = lax.fori_loop(0, GROUPS - 1, _compact, jnp.int32(0))

    tbase = (GROUPS - 1) * 16
    tslots = plsc.load_gather(map_v, [dst_v[pl.ds(tbase, 16)]])
    thit = (tslots >= 0) & (lanes < E_PER_W - tbase)
    plsc.store_compressed(hit_slot.at[pl.ds(cnt, 16)], tslots, mask=thit)
    plsc.store_compressed(hit_src.at[pl.ds(cnt, 16)],
                          src_v[pl.ds(tbase, 16)], mask=thit)
    cnt = cnt + plsc.all_reduce_population_count(thit)[0]

    trash = jnp.full((16,), TRASH, _i32)
    for j in range(FIRE // 16):
        hit_slot[pl.ds(cnt + j * 16, 16)] = trash
        hit_src[pl.ds(cnt + j * 16, 16)] = zeros16

    n_chunks = (cnt + FIRE - 1) // FIRE
    rows = (rows_v, rows_v2)
    src_f = (src_fire, src_fire2)
    slot_f = (slot_fire, slot_fire2)
    gsem = (gsem0, gsem1)
    ssem = (ssem0, ssem1)
    dsem = (dsem0, dsem1)

    def _ga(p):
        return pltpu.make_async_copy(
            x_hbm.at[plsc.Indices(src_f[p])], rows[p], gsem[p])

    def _sa(p):
        return pltpu.make_async_copy(
            rows[p], agg_sh.at[plsc.Indices(slot_f[p])], ssem[p])

    def _da(p):
        return pltpu.make_async_copy(
            ones_v, deg_sh.at[plsc.Indices(slot_f[p])], dsem[p])

    def _load_idx(p, k):
        base = k * FIRE
        for j in range(FIRE // 16):
            src_f[p][pl.ds(j * 16, 16)] = hit_src[pl.ds(base + j * 16, 16)]
            slot_f[p][pl.ds(j * 16, 16)] = hit_slot[pl.ds(base + j * 16, 16)]

    @pl.when(n_chunks > 0)
    def _():
        _load_idx(0, 0)
        _ga(0).start()
        _da(0).start(add=True)

    def _step(k, p):
        def _one(p):
            q = 1 - p
            _ga(p).wait()
            _sa(p).start(add=True)

            @pl.when(k + 1 < n_chunks)
            def _():
                @pl.when(k >= 1)
                def _():
                    _sa(q).wait()
                    _da(q).wait()
                _load_idx(q, k + 1)
                _ga(q).start()
                _da(q).start(add=True)

        @pl.when(p == 0)
        def _():
            _one(0)

        @pl.when(p == 1)
        def _():
            _one(1)

        return 1 - p

    lax.fori_loop(0, n_chunks, _step, jnp.int32(0))

    last = n_chunks - 1

    @pl.when(n_chunks >= 2)
    def _():
        p = (last - 1) % 2

        @pl.when(p == 0)
        def _():
            _sa(0).wait()
            _da(0).wait()

        @pl.when(p == 1)
        def _():
            _sa(1).wait()
            _da(1).wait()

    @pl.when(n_chunks >= 1)
    def _():
        p = last % 2

        @pl.when(p == 0)
        def _():
            _sa(0).wait()
            _da(0).wait()

        @pl.when(p == 1)
        def _():
            _sa(1).wait()
            _da(1).wait()

    plsc.subcore_barrier()

    RPW = NB // NS
    co_a = pltpu.make_async_copy(agg_sh.at[pl.ds(sid * RPW, RPW)],
                                 agg_out.at[cid, pl.ds(sid * RPW, RPW)],
                                 gsem0)
    co_d = pltpu.make_async_copy(deg_sh.at[pl.ds(sid * RPW, RPW)],
                                 deg_out.at[cid, pl.ds(sid * RPW, RPW)],
                                 gsem1)
    co_a.start()
    co_d.start()
    co_a.wait()
    co_d.wait()


@functools.partial(
    pl.kernel,
    out_type=(
        jax.ShapeDtypeStruct((NC, NB, D), _f32),
        jax.ShapeDtypeStruct((NC, NB, 16), _f32),
        jax.ShapeDtypeStruct((NB, D), _f32),
        jax.ShapeDtypeStruct((NB,), _i32),
    ),
    mesh=_MESH,
    compiler_params=_SC_PARAMS,
    scratch_types=[
        pltpu.VMEM((MAPN,), _i32),
        pltpu.VMEM((NB,), _i32),
        pltpu.VMEM((E_PER_W + 8,), _i32),
        pltpu.VMEM((E_PER_W + 8,), _i32),
        pltpu.VMEM((HITCAP,), _i32),
        pltpu.VMEM((HITCAP,), _i32),
        pltpu.VMEM((FIRE,), _i32),
        pltpu.VMEM((FIRE,), _i32),
        pltpu.VMEM((FIRE,), _i32),
        pltpu.VMEM((FIRE,), _i32),
        pltpu.VMEM((FIRE, D), _f32),
        pltpu.VMEM((FIRE, D), _f32),
        pltpu.VMEM((FIRE, 16), _f32),
        pltpu.VMEM((NB_PER_W,), _i32),
        pltpu.SemaphoreType.DMA,
        pltpu.SemaphoreType.DMA,
        pltpu.SemaphoreType.DMA,
        pltpu.SemaphoreType.DMA,
        pltpu.SemaphoreType.DMA,
        pltpu.SemaphoreType.DMA,
        pltpu.VMEM_SHARED((AGG_ROWS, D), _f32),
        pltpu.VMEM_SHARED((AGG_ROWS, 16), _f32),
    ],
)
def _edge_kernel(*refs):
    _edge_body(*refs)


def _dense_body(xb_ref, agg_ref, deg_ref, ws_ref, wn_ref, h_ref):
    agg = agg_ref[0] + agg_ref[1]
    deg = jnp.sum(deg_ref[0] + deg_ref[1], axis=1, keepdims=True)
    mean = agg / jnp.maximum(deg, 1.0)
    h = jnp.dot(xb_ref[...], ws_ref[...],
                preferred_element_type=_f32)
    h = h + jnp.dot(mean, wn_ref[...],
                    preferred_element_type=_f32)
    h = jnp.maximum(h, 0.0)
    norm = jnp.sqrt(jnp.sum(h * h, axis=1, keepdims=True))
    h_ref[...] = h / jnp.maximum(norm, 1e-12)


def _dense_call(xb, agg, deg, w_self, w_neigh):
    return pl.pallas_call(
        _dense_body,
        out_shape=jax.ShapeDtypeStruct((NB, D), _f32),
    )(xb, agg, deg, w_self, w_neigh)


def _out_body(h_hbm, perm_hbm, e1_out, e2_out, idx_v, rows_v):
    wid = lax.axis_index("c") * NS + lax.axis_index("s")
    pltpu.sync_copy(perm_hbm.at[pl.ds(wid * NB_PER_W, NB_PER_W)], idx_v)
    pltpu.sync_copy(h_hbm.at[plsc.Indices(idx_v)], rows_v)

    @pl.when(wid < NS)
    def _():
        pltpu.sync_copy(rows_v, e1_out.at[pl.ds(wid * NB_PER_W, NB_PER_W)])

    @pl.when(wid >= NS)
    def _():
        pltpu.sync_copy(rows_v,
                        e2_out.at[pl.ds((wid - NS) * NB_PER_W, NB_PER_W)])


@functools.partial(
    pl.kernel,
    out_type=(
        jax.ShapeDtypeStruct((B, D), _f32),
        jax.ShapeDtypeStruct((B, D), _f32),
    ),
    mesh=_MESH,
    compiler_params=_SC_PARAMS,
    scratch_types=[
        pltpu.VMEM((NB_PER_W,), _i32),
        pltpu.VMEM((NB_PER_W, D), _f32),
    ],
)
def _out_kernel(h_hbm, perm_hbm, e1_out, e2_out, idx_v, rows_v):
    _out_body(h_hbm, perm_hbm, e1_out, e2_out, idx_v, rows_v)


def kernel(x, edge_index, nodes1, nodes2, W_self, W_neigh):
    agg, deg, xb, perm = _edge_kernel(x, edge_index, nodes1, nodes2)
    h = _dense_call(xb, agg, deg, W_self, W_neigh)
    e1, e2 = _out_kernel(h, perm)
    return (e1, e2)
